# Initial kernel scaffold; baseline (speedup 1.0000x reference)
import jax
import jax.numpy as jnp
from jax.experimental import pallas as pl


def kernel(x, e, edge_index, qkv_w, qkv_b, ew_w, ew_b, aw, c1_w, c1_b, c2_w, c2_b, f1_w, f1_b, f2_w, f2_b, ln1h_w, ln1h_b, ln2h_w, ln2h_b, ln1c_w, ln1c_b, ln2c_w, ln2c_b):
    raise NotImplementedError("write your pallas kernel here")


# speedup vs baseline: 22.0843x; 22.0843x over previous
import functools

import jax
import jax.numpy as jnp
from jax import lax
from jax.experimental import pallas as pl
from jax.experimental.pallas import tpu as pltpu
from jax.experimental.pallas import tpu_sc as plsc

_N = 10000
_E = 320000
_HID = 128
_HEADS = 8
_DH = 16
_CLAMP = 5.0
_EPS = 1e-5

_NC = 2
_NS = 16
_NW = _NC * _NS
_PERW = _E // _NW
_CG = 80
_NCHG = _PERW // _CG
_EDW = 144
_HN = _N // _NC
_ACC_R = 5120
_ROWS_T = _ACC_R // _NS
_PERW_S = _E // _NS
_NCHS = _PERW_S // _CG

_NB = 1000
_EB = 2000


def _ln_tc(h, w, b):
    mu = jnp.mean(h, axis=-1, keepdims=True)
    var = jnp.mean((h - mu) ** 2, axis=-1, keepdims=True)
    return (h - mu) * jax.lax.rsqrt(var + _EPS) * w + b


def _qkv_body(x_ref, wT_ref, b_ref, q_ref, k_ref, v_ref):
    y = jnp.dot(x_ref[...], wT_ref[...], preferred_element_type=jnp.float32)
    y = y + b_ref[...]
    q_ref[...] = y[:, :_HID]
    k_ref[...] = y[:, _HID:2 * _HID]
    v_ref[...] = y[:, 2 * _HID:]


def _qkv_call(x, qkvT, qkvb):
    return pl.pallas_call(
        _qkv_body,
        grid=(_N // _NB,),
        in_specs=[
            pl.BlockSpec((_NB, _HID), lambda i: (i, 0)),
            pl.BlockSpec((_HID, 3 * _HID), lambda i: (0, 0)),
            pl.BlockSpec((1, 3 * _HID), lambda i: (0, 0)),
        ],
        out_specs=[pl.BlockSpec((_NB, _HID), lambda i: (i, 0))] * 3,
        out_shape=[jax.ShapeDtypeStruct((_N, _HID), jnp.float32)] * 3,
    )(x, qkvT, qkvb)


def _gather_body(q_hbm, k_hbm, v_hbm, dst_hbm, src_hbm, qk_hbm, vg_hbm,
                 idxd, idxs, qb, kb, vb, sem):
    wid = lax.axis_index("s") * _NC + lax.axis_index("c")
    base = wid * _PERW

    def chunk(j, carry):
        off = base + j * _CG
        pltpu.sync_copy(dst_hbm.at[pl.ds(off, _CG)], idxd)
        pltpu.sync_copy(src_hbm.at[pl.ds(off, _CG)], idxs)
        cq = pltpu.async_copy(q_hbm.at[idxd], qb, sem)
        ck = pltpu.async_copy(k_hbm.at[idxs], kb, sem)
        cv = pltpu.async_copy(v_hbm.at[idxs], vb, sem)
        cq.wait()
        ck.wait()
        cv.wait()

        def addrow(r, c2):
            for cc in range(_HID // 16):
                sl = pl.ds(cc * 16, 16)
                qb[r, sl] = qb[r, sl] + kb[r, sl]
            return c2

        lax.fori_loop(0, _CG, addrow, 0)
        pltpu.sync_copy(qb, qk_hbm.at[pl.ds(off, _CG)])
        pltpu.sync_copy(vb, vg_hbm.at[pl.ds(off, _CG)])
        return carry

    lax.fori_loop(0, _NCHG, chunk, 0)


def _gather_call(q, k, v, dst, src):
    f = pl.kernel(
        _gather_body,
        out_type=[jax.ShapeDtypeStruct((_E, _HID), jnp.float32)] * 2,
        mesh=plsc.VectorSubcoreMesh(core_axis_name="c", subcore_axis_name="s"),
        scratch_types=[
            pltpu.VMEM((_CG,), jnp.int32),
            pltpu.VMEM((_CG,), jnp.int32),
            pltpu.VMEM((_CG, _HID), jnp.float32),
            pltpu.VMEM((_CG, _HID), jnp.float32),
            pltpu.VMEM((_CG, _HID), jnp.float32),
            pltpu.SemaphoreType.DMA,
        ],
    )
    return f(q, k, v, dst, src)


def _edge_body(e_ref, qk_ref, vg_ref, ewT, ewb, A_ref, c1T, c1b, c2T, c2b,
               S_ref, l1w, l1b, l2w, l2b, ed_ref, co_ref):
    e_blk = e_ref[...]
    eh = jnp.dot(e_blk, ewT[...], preferred_element_type=jnp.float32)
    eh = eh + ewb[...]
    Ew = eh[:, :_HID]
    Eb = eh[:, _HID:]
    conn1 = qk_ref[...] * Ew
    a = jnp.abs(conn1)
    conn2 = (jnp.sign(conn1) * jnp.sqrt(jnp.where(a > 0, a, 1.0))
             * (a > 0).astype(jnp.float32))
    conn = jnp.maximum(conn2 + Eb, 0.0)
    score = jnp.dot(conn, A_ref[...], preferred_element_type=jnp.float32)
    w = jnp.exp(jnp.clip(score, -_CLAMP, _CLAMP))
    cc1 = jnp.dot(conn, c1T[...], preferred_element_type=jnp.float32)
    cc1 = cc1 + c1b[...]
    msg = vg_ref[...] + cc1
    wfull = jnp.dot(w, S_ref[...], preferred_element_type=jnp.float32)
    ed_ref[:, :_HID] = msg * wfull
    ed_ref[:, _HID:_HID + _HEADS] = w
    ed_ref[:, _HID + _HEADS:] = jnp.zeros(
        (e_blk.shape[0], _EDW - _HID - _HEADS), jnp.float32)
    co = _ln_tc(cc1, l1w[...], l1b[...])
    co = jnp.maximum(co, 0.0)
    co = jnp.dot(co, c2T[...], preferred_element_type=jnp.float32)
    co = co + c2b[...] + e_blk
    co_ref[...] = _ln_tc(co, l2w[...], l2b[...])


def _edge_call(e, qkg, vg, ewT, ewb, A, c1T, c1b, c2T, c2b, S,
               l1w, l1b, l2w, l2b):
    full = lambda shape: pl.BlockSpec(shape, lambda i: (0, 0))
    return pl.pallas_call(
        _edge_body,
        grid=(_E // _EB,),
        in_specs=[
            pl.BlockSpec((_EB, _HID), lambda i: (i, 0)),
            pl.BlockSpec((_EB, _HID), lambda i: (i, 0)),
            pl.BlockSpec((_EB, _HID), lambda i: (i, 0)),
            full((_HID, 2 * _HID)),
            full((1, 2 * _HID)),
            full((_HID, _HEADS)),
            full((_HID, _HID)),
            full((1, _HID)),
            full((_HID, _HID)),
            full((1, _HID)),
            full((_HEADS, _HID)),
            full((1, _HID)),
            full((1, _HID)),
            full((1, _HID)),
            full((1, _HID)),
        ],
        out_specs=[
            pl.BlockSpec((_EB, _EDW), lambda i: (i, 0)),
            pl.BlockSpec((_EB, _HID), lambda i: (i, 0)),
        ],
        out_shape=[
            jax.ShapeDtypeStruct((_E, _EDW), jnp.float32),
            jax.ShapeDtypeStruct((_E, _HID), jnp.float32),
        ],
    )(e, qkg, vg, ewT, ewb, A, c1T, c1b, c2T, c2b, S, l1w, l1b, l2w, l2b)


def _scatter_body(ed_hbm, dst_hbm, out_hbm, idxv, rows, zb, acc):
    cid = lax.axis_index("c")
    sid = lax.axis_index("s")
    z16 = jnp.zeros((16,), jnp.float32)

    def zrow(r, c):
        for cc in range(_EDW // 16):
            zb[r, pl.ds(cc * 16, 16)] = z16
        return c

    lax.fori_loop(0, _ROWS_T, zrow, 0)
    pltpu.sync_copy(zb, acc.at[pl.ds(sid * _ROWS_T, _ROWS_T)])
    plsc.subcore_barrier()

    lo = cid * _HN
    trash = _HN + sid
    base = sid * _PERW_S

    def chunk(j, carry):
        off = base + j * _CG
        pltpu.sync_copy(dst_hbm.at[pl.ds(off, _CG)], idxv)
        pltpu.sync_copy(ed_hbm.at[pl.ds(off, _CG)], rows)
        for s in range(_CG // 16):
            sl = pl.ds(s * 16, 16)
            v = idxv[sl]
            local = v - lo
            ok = (local >= 0) & (local < _HN)
            idxv[sl] = jnp.where(ok, local, trash)
        pltpu.sync_copy(rows, acc.at[idxv], add=True)
        return carry

    lax.fori_loop(0, _NCHS, chunk, 0)
    plsc.subcore_barrier()
    pltpu.sync_copy(acc.at[pl.ds(sid * _ROWS_T, _ROWS_T)],
                    out_hbm.at[cid, pl.ds(sid * _ROWS_T, _ROWS_T)])


def _scatter_call(ed, dst):
    f = pl.kernel(
        _scatter_body,
        out_type=jax.ShapeDtypeStruct((_NC, _ACC_R, _EDW), jnp.float32),
        mesh=plsc.VectorSubcoreMesh(core_axis_name="c", subcore_axis_name="s"),
        scratch_types=[
            pltpu.VMEM((_CG,), jnp.int32),
            pltpu.VMEM((_CG, _EDW), jnp.float32),
            pltpu.VMEM((_ROWS_T, _EDW), jnp.float32),
            pltpu.VMEM_SHARED((_ACC_R, _EDW), jnp.float32),
        ],
        compiler_params=pltpu.CompilerParams(use_tc_tiling_on_sc=False),
    )
    return f(ed, dst)


def _node_body(x_ref, p_ref, S_ref, f1T, f1b, f2T, f2b, l1w, l1b, l2w, l2b,
               h_ref):
    p = p_ref[0]
    aggW = p[:, :_HID]
    sumw = p[:, _HID:_HID + _HEADS]
    inv = 1.0 / (sumw + 1e-16)
    agg = aggW * jnp.dot(inv, S_ref[...], preferred_element_type=jnp.float32)
    h0 = x_ref[...] + agg
    h = _ln_tc(h0, l1w[...], l1b[...])
    h = jnp.dot(h, f1T[...], preferred_element_type=jnp.float32) + f1b[...]
    h = jnp.maximum(h, 0.0)
    h = jnp.dot(h, f2T[...], preferred_element_type=jnp.float32) + f2b[...]
    h = h + h0
    h_ref[...] = _ln_tc(h, l2w[...], l2b[...])


def _node_call(x, parts, S, f1T, f1b, f2T, f2b, l1w, l1b, l2w, l2b):
    full = lambda shape: pl.BlockSpec(shape, lambda i: (0,) * len(shape))
    return pl.pallas_call(
        _node_body,
        grid=(_N // _NB,),
        in_specs=[
            pl.BlockSpec((_NB, _HID), lambda i: (i, 0)),
            pl.BlockSpec((1, _NB, _EDW),
                         lambda i: (i // (_HN // _NB), i % (_HN // _NB), 0)),
            full((_HEADS, _HID)),
            full((_HID, 2 * _HID)),
            full((1, 2 * _HID)),
            full((2 * _HID, _HID)),
            full((1, _HID)),
            full((1, _HID)),
            full((1, _HID)),
            full((1, _HID)),
            full((1, _HID)),
        ],
        out_specs=pl.BlockSpec((_NB, _HID), lambda i: (i, 0)),
        out_shape=jax.ShapeDtypeStruct((_N, _HID), jnp.float32),
    )(x, parts, S, f1T, f1b, f2T, f2b, l1w, l1b, l2w, l2b)


def kernel(x, e, edge_index, qkv_w, qkv_b, ew_w, ew_b, aw, c1_w, c1_b,
           c2_w, c2_b, f1_w, f1_b, f2_w, f2_b, ln1h_w, ln1h_b, ln2h_w,
           ln2h_b, ln1c_w, ln1c_b, ln2c_w, ln2c_b):
    f32 = jnp.float32
    dst = edge_index[0]
    src = edge_index[1]
    row = lambda b: b.reshape(1, -1).astype(f32)

    S = jnp.repeat(jnp.eye(_HEADS, dtype=f32), _DH, axis=1)
    awflat = aw[:, :, 0].T.reshape(_HID, 1)
    A = S.T * awflat

    q, k, v = _qkv_call(x, qkv_w.T, row(qkv_b))
    qkg, vg = _gather_call(q, k, v, dst, src)
    ed, co = _edge_call(e, qkg, vg, ew_w.T, row(ew_b), A, c1_w.T, row(c1_b),
                        c2_w.T, row(c2_b), S, row(ln1c_w), row(ln1c_b),
                        row(ln2c_w), row(ln2c_b))
    parts = _scatter_call(ed, dst)
    h = _node_call(x, parts, S, f1_w.T, row(f1_b), f2_w.T, row(f2_b),
                   row(ln1h_w), row(ln1h_b), row(ln2h_w), row(ln2h_b))
    return h, co


# speedup vs baseline: 29.1805x; 1.3213x over previous
import functools

import jax
import jax.numpy as jnp
from jax import lax
from jax.experimental import pallas as pl
from jax.experimental.pallas import tpu as pltpu
from jax.experimental.pallas import tpu_sc as plsc

_N = 10000
_E = 320000
_HID = 128
_HEADS = 8
_DH = 16
_CLAMP = 5.0
_EPS = 1e-5

_NC = 2
_NS = 16
_NW = _NC * _NS
_PERW = _E // _NW
_CG = 80
_NCHG = _PERW // _CG
_EDW = 144
_HN = _N // _NC
_ACC_R = 5120
_ROWS_T = _ACC_R // _NS
_PERW_S = _E // _NS
_NCHS = _PERW_S // _CG

_NB = 1000
_EB = 2000


def _ln_tc(h, w, b):
    mu = jnp.mean(h, axis=-1, keepdims=True)
    var = jnp.mean((h - mu) ** 2, axis=-1, keepdims=True)
    return (h - mu) * jax.lax.rsqrt(var + _EPS) * w + b


def _qkv_body(x_ref, wT_ref, b_ref, q_ref, k_ref, v_ref):
    y = jnp.dot(x_ref[...], wT_ref[...], preferred_element_type=jnp.float32)
    y = y + b_ref[...]
    q_ref[...] = y[:, :_HID]
    k_ref[...] = y[:, _HID:2 * _HID]
    v_ref[...] = y[:, 2 * _HID:]


def _qkv_call(x, qkvT, qkvb):
    return pl.pallas_call(
        _qkv_body,
        grid=(_N // _NB,),
        in_specs=[
            pl.BlockSpec((_NB, _HID), lambda i: (i, 0)),
            pl.BlockSpec((_HID, 3 * _HID), lambda i: (0, 0)),
            pl.BlockSpec((1, 3 * _HID), lambda i: (0, 0)),
        ],
        out_specs=[pl.BlockSpec((_NB, _HID), lambda i: (i, 0))] * 3,
        out_shape=[jax.ShapeDtypeStruct((_N, _HID), jnp.float32)] * 3,
    )(x, qkvT, qkvb)


def _gather_body(q_hbm, k_hbm, v_hbm, dst_hbm, src_hbm, qk_hbm, vg_hbm,
                 idxd0, idxs0, qb0, kb0, vb0, sem0,
                 idxd1, idxs1, qb1, kb1, vb1, sem1):
    bufs = ((idxd0, idxs0, qb0, kb0, vb0, sem0),
            (idxd1, idxs1, qb1, kb1, vb1, sem1))
    wid = lax.axis_index("s") * _NC + lax.axis_index("c")
    base = wid * _PERW

    def issue(j, buf):
        idxd, idxs, qb, kb, vb, sem = buf
        off = base + j * _CG
        pltpu.sync_copy(dst_hbm.at[pl.ds(off, _CG)], idxd)
        pltpu.sync_copy(src_hbm.at[pl.ds(off, _CG)], idxs)
        pltpu.async_copy(q_hbm.at[idxd], qb, sem)
        pltpu.async_copy(k_hbm.at[idxs], kb, sem)
        pltpu.async_copy(v_hbm.at[idxs], vb, sem)

    def finish(j, buf):
        idxd, idxs, qb, kb, vb, sem = buf
        off = base + j * _CG
        pltpu.make_async_copy(q_hbm.at[idxd], qb, sem).wait()
        pltpu.make_async_copy(k_hbm.at[idxs], kb, sem).wait()
        pltpu.make_async_copy(v_hbm.at[idxs], vb, sem).wait()

        def addrow(r, c2):
            for cc in range(_HID // 16):
                sl = pl.ds(cc * 16, 16)
                qb[r, sl] = qb[r, sl] + kb[r, sl]
            return c2

        lax.fori_loop(0, _CG, addrow, 0)
        pltpu.sync_copy(qb, qk_hbm.at[pl.ds(off, _CG)])
        pltpu.sync_copy(vb, vg_hbm.at[pl.ds(off, _CG)])

    issue(0, bufs[0])

    def outer(g, carry):
        j0 = 2 * g
        issue(j0 + 1, bufs[1])
        finish(j0, bufs[0])
        issue(j0 + 2, bufs[0])
        finish(j0 + 1, bufs[1])
        return carry

    lax.fori_loop(0, _NCHG // 2, outer, 0)
    finish(_NCHG - 1, bufs[0])


def _gather_call(q, k, v, dst, src):
    buf_set = [
        pltpu.VMEM((_CG,), jnp.int32),
        pltpu.VMEM((_CG,), jnp.int32),
        pltpu.VMEM((_CG, _HID), jnp.float32),
        pltpu.VMEM((_CG, _HID), jnp.float32),
        pltpu.VMEM((_CG, _HID), jnp.float32),
        pltpu.SemaphoreType.DMA,
    ]
    f = pl.kernel(
        _gather_body,
        out_type=[jax.ShapeDtypeStruct((_E, _HID), jnp.float32)] * 2,
        mesh=plsc.VectorSubcoreMesh(core_axis_name="c", subcore_axis_name="s"),
        scratch_types=buf_set + buf_set,
    )
    return f(q, k, v, dst, src)


def _edge_body(e_ref, qk_ref, vg_ref, ewT, ewb, A_ref, c1T, c1b, c2T, c2b,
               S_ref, l1w, l1b, l2w, l2b, ed_ref, co_ref):
    e_blk = e_ref[...]
    eh = jnp.dot(e_blk, ewT[...], preferred_element_type=jnp.float32)
    eh = eh + ewb[...]
    Ew = eh[:, :_HID]
    Eb = eh[:, _HID:]
    conn1 = qk_ref[...] * Ew
    a = jnp.abs(conn1)
    conn2 = (jnp.sign(conn1) * jnp.sqrt(jnp.where(a > 0, a, 1.0))
             * (a > 0).astype(jnp.float32))
    conn = jnp.maximum(conn2 + Eb, 0.0)
    score = jnp.dot(conn, A_ref[...], preferred_element_type=jnp.float32)
    w = jnp.exp(jnp.clip(score, -_CLAMP, _CLAMP))
    cc1 = jnp.dot(conn, c1T[...], preferred_element_type=jnp.float32)
    cc1 = cc1 + c1b[...]
    msg = vg_ref[...] + cc1
    wfull = jnp.dot(w, S_ref[...], preferred_element_type=jnp.float32)
    ed_ref[:, :_HID] = msg * wfull
    ed_ref[:, _HID:_HID + _HEADS] = w
    ed_ref[:, _HID + _HEADS:] = jnp.zeros(
        (e_blk.shape[0], _EDW - _HID - _HEADS), jnp.float32)
    co = _ln_tc(cc1, l1w[...], l1b[...])
    co = jnp.maximum(co, 0.0)
    co = jnp.dot(co, c2T[...], preferred_element_type=jnp.float32)
    co = co + c2b[...] + e_blk
    co_ref[...] = _ln_tc(co, l2w[...], l2b[...])


def _edge_call(e, qkg, vg, ewT, ewb, A, c1T, c1b, c2T, c2b, S,
               l1w, l1b, l2w, l2b):
    full = lambda shape: pl.BlockSpec(shape, lambda i: (0, 0))
    return pl.pallas_call(
        _edge_body,
        grid=(_E // _EB,),
        in_specs=[
            pl.BlockSpec((_EB, _HID), lambda i: (i, 0)),
            pl.BlockSpec((_EB, _HID), lambda i: (i, 0)),
            pl.BlockSpec((_EB, _HID), lambda i: (i, 0)),
            full((_HID, 2 * _HID)),
            full((1, 2 * _HID)),
            full((_HID, _HEADS)),
            full((_HID, _HID)),
            full((1, _HID)),
            full((_HID, _HID)),
            full((1, _HID)),
            full((_HEADS, _HID)),
            full((1, _HID)),
            full((1, _HID)),
            full((1, _HID)),
            full((1, _HID)),
        ],
        out_specs=[
            pl.BlockSpec((_EB, _EDW), lambda i: (i, 0)),
            pl.BlockSpec((_EB, _HID), lambda i: (i, 0)),
        ],
        out_shape=[
            jax.ShapeDtypeStruct((_E, _EDW), jnp.float32),
            jax.ShapeDtypeStruct((_E, _HID), jnp.float32),
        ],
    )(e, qkg, vg, ewT, ewb, A, c1T, c1b, c2T, c2b, S, l1w, l1b, l2w, l2b)


def _scatter_body(ed_hbm, dst_hbm, out_hbm, idx0, rows0, sem0,
                  idx1, rows1, sem1, zb, acc):
    bufs = ((idx0, rows0, sem0), (idx1, rows1, sem1))
    cid = lax.axis_index("c")
    sid = lax.axis_index("s")
    z16 = jnp.zeros((16,), jnp.float32)

    def zrow(r, c):
        for cc in range(_EDW // 16):
            zb[r, pl.ds(cc * 16, 16)] = z16
        return c

    lax.fori_loop(0, _ROWS_T, zrow, 0)
    pltpu.sync_copy(zb, acc.at[pl.ds(sid * _ROWS_T, _ROWS_T)])
    plsc.subcore_barrier()

    lo = cid * _HN
    trash = _HN + sid
    base = sid * _PERW_S

    def issue(j, buf):
        idxv, rows, sem = buf
        off = base + j * _CG
        pltpu.sync_copy(dst_hbm.at[pl.ds(off, _CG)], idxv)
        for s in range(_CG // 16):
            sl = pl.ds(s * 16, 16)
            v = idxv[sl]
            local = v - lo
            ok = (local >= 0) & (local < _HN)
            idxv[sl] = jnp.where(ok, local, trash)
        pltpu.async_copy(ed_hbm.at[pl.ds(off, _CG)], rows, sem)

    def finish(j, buf):
        idxv, rows, sem = buf
        off = base + j * _CG
        pltpu.make_async_copy(ed_hbm.at[pl.ds(off, _CG)], rows, sem).wait()
        pltpu.sync_copy(rows, acc.at[idxv], add=True)

    issue(0, bufs[0])

    def outer(g, carry):
        j0 = 2 * g
        issue(j0 + 1, bufs[1])
        finish(j0, bufs[0])
        issue(j0 + 2, bufs[0])
        finish(j0 + 1, bufs[1])
        return carry

    lax.fori_loop(0, _NCHS // 2 - 1, outer, 0)
    issue(_NCHS - 1, bufs[1])
    finish(_NCHS - 2, bufs[0])
    finish(_NCHS - 1, bufs[1])
    plsc.subcore_barrier()
    pltpu.sync_copy(acc.at[pl.ds(sid * _ROWS_T, _ROWS_T)],
                    out_hbm.at[cid, pl.ds(sid * _ROWS_T, _ROWS_T)])


def _scatter_call(ed, dst):
    buf_set = [
        pltpu.VMEM((_CG,), jnp.int32),
        pltpu.VMEM((_CG, _EDW), jnp.float32),
        pltpu.SemaphoreType.DMA,
    ]
    f = pl.kernel(
        _scatter_body,
        out_type=jax.ShapeDtypeStruct((_NC, _ACC_R, _EDW), jnp.float32),
        mesh=plsc.VectorSubcoreMesh(core_axis_name="c", subcore_axis_name="s"),
        scratch_types=buf_set + buf_set + [
            pltpu.VMEM((_ROWS_T, _EDW), jnp.float32),
            pltpu.VMEM_SHARED((_ACC_R, _EDW), jnp.float32),
        ],
        compiler_params=pltpu.CompilerParams(use_tc_tiling_on_sc=False),
    )
    return f(ed, dst)


def _node_body(x_ref, p_ref, S_ref, f1T, f1b, f2T, f2b, l1w, l1b, l2w, l2b,
               h_ref):
    p = p_ref[0]
    aggW = p[:, :_HID]
    sumw = p[:, _HID:_HID + _HEADS]
    inv = 1.0 / (sumw + 1e-16)
    agg = aggW * jnp.dot(inv, S_ref[...], preferred_element_type=jnp.float32)
    h0 = x_ref[...] + agg
    h = _ln_tc(h0, l1w[...], l1b[...])
    h = jnp.dot(h, f1T[...], preferred_element_type=jnp.float32) + f1b[...]
    h = jnp.maximum(h, 0.0)
    h = jnp.dot(h, f2T[...], preferred_element_type=jnp.float32) + f2b[...]
    h = h + h0
    h_ref[...] = _ln_tc(h, l2w[...], l2b[...])


def _node_call(x, parts, S, f1T, f1b, f2T, f2b, l1w, l1b, l2w, l2b):
    full = lambda shape: pl.BlockSpec(shape, lambda i: (0,) * len(shape))
    return pl.pallas_call(
        _node_body,
        grid=(_N // _NB,),
        in_specs=[
            pl.BlockSpec((_NB, _HID), lambda i: (i, 0)),
            pl.BlockSpec((1, _NB, _EDW),
                         lambda i: (i // (_HN // _NB), i % (_HN // _NB), 0)),
            full((_HEADS, _HID)),
            full((_HID, 2 * _HID)),
            full((1, 2 * _HID)),
            full((2 * _HID, _HID)),
            full((1, _HID)),
            full((1, _HID)),
            full((1, _HID)),
            full((1, _HID)),
            full((1, _HID)),
        ],
        out_specs=pl.BlockSpec((_NB, _HID), lambda i: (i, 0)),
        out_shape=jax.ShapeDtypeStruct((_N, _HID), jnp.float32),
    )(x, parts, S, f1T, f1b, f2T, f2b, l1w, l1b, l2w, l2b)


def kernel(x, e, edge_index, qkv_w, qkv_b, ew_w, ew_b, aw, c1_w, c1_b,
           c2_w, c2_b, f1_w, f1_b, f2_w, f2_b, ln1h_w, ln1h_b, ln2h_w,
           ln2h_b, ln1c_w, ln1c_b, ln2c_w, ln2c_b):
    f32 = jnp.float32
    dst = edge_index[0]
    src = edge_index[1]
    row = lambda b: b.reshape(1, -1).astype(f32)

    S = jnp.repeat(jnp.eye(_HEADS, dtype=f32), _DH, axis=1)
    awflat = aw[:, :, 0].T.reshape(_HID, 1)
    A = S.T * awflat

    q, k, v = _qkv_call(x, qkv_w.T, row(qkv_b))
    qkg, vg = _gather_call(q, k, v, dst, src)
    ed, co = _edge_call(e, qkg, vg, ew_w.T, row(ew_b), A, c1_w.T, row(c1_b),
                        c2_w.T, row(c2_b), S, row(ln1c_w), row(ln1c_b),
                        row(ln2c_w), row(ln2c_b))
    parts = _scatter_call(ed, dst)
    h = _node_call(x, parts, S, f1_w.T, row(f1_b), f2_w.T, row(f2_b),
                   row(ln1h_w), row(ln1h_b), row(ln2h_w), row(ln2h_b))
    return h, co


# speedup vs baseline: 29.6580x; 1.0164x over previous
import functools

import jax
import jax.numpy as jnp
from jax import lax
from jax.experimental import pallas as pl
from jax.experimental.pallas import tpu as pltpu
from jax.experimental.pallas import tpu_sc as plsc

_N = 10000
_E = 320000
_HID = 128
_HEADS = 8
_DH = 16
_CLAMP = 5.0
_EPS = 1e-5

_NC = 2
_NS = 16
_NW = _NC * _NS
_PERW = _E // _NW
_CG = 80
_NCHG = _PERW // _CG
_EDW = 144
_HN = _N // _NC
_ACC_R = 5120
_ROWS_T = _ACC_R // _NS
_PERW_S = _E // _NS
_NCHS = _PERW_S // _CG

_NB = 1000
_EB = 2000


def _ln_tc(h, w, b):
    mu = jnp.mean(h, axis=-1, keepdims=True)
    var = jnp.mean((h - mu) ** 2, axis=-1, keepdims=True)
    return (h - mu) * jax.lax.rsqrt(var + _EPS) * w + b


def _gather_body(x_hbm, dst_hbm, src_hbm, xd_hbm, xs_hbm,
                 idxd0, idxs0, db0, sb0, sem0,
                 idxd1, idxs1, db1, sb1, sem1):
    bufs = ((idxd0, idxs0, db0, sb0, sem0),
            (idxd1, idxs1, db1, sb1, sem1))
    wid = lax.axis_index("s") * _NC + lax.axis_index("c")
    base = wid * _PERW

    def issue(j, buf):
        idxd, idxs, db, sb, sem = buf
        off = base + j * _CG
        pltpu.sync_copy(dst_hbm.at[pl.ds(off, _CG)], idxd)
        pltpu.sync_copy(src_hbm.at[pl.ds(off, _CG)], idxs)
        pltpu.async_copy(x_hbm.at[idxd], db, sem)
        pltpu.async_copy(x_hbm.at[idxs], sb, sem)

    def finish(j, buf):
        idxd, idxs, db, sb, sem = buf
        off = base + j * _CG
        pltpu.make_async_copy(x_hbm.at[idxd], db, sem).wait()
        pltpu.make_async_copy(x_hbm.at[idxs], sb, sem).wait()
        pltpu.sync_copy(db, xd_hbm.at[pl.ds(off, _CG)])
        pltpu.sync_copy(sb, xs_hbm.at[pl.ds(off, _CG)])

    issue(0, bufs[0])

    def outer(g, carry):
        j0 = 2 * g
        issue(j0 + 1, bufs[1])
        finish(j0, bufs[0])
        issue(j0 + 2, bufs[0])
        finish(j0 + 1, bufs[1])
        return carry

    lax.fori_loop(0, _NCHG // 2, outer, 0)
    finish(_NCHG - 1, bufs[0])


def _gather_call(x, dst, src):
    buf_set = [
        pltpu.VMEM((_CG,), jnp.int32),
        pltpu.VMEM((_CG,), jnp.int32),
        pltpu.VMEM((_CG, _HID), jnp.float32),
        pltpu.VMEM((_CG, _HID), jnp.float32),
        pltpu.SemaphoreType.DMA,
    ]
    f = pl.kernel(
        _gather_body,
        out_type=[jax.ShapeDtypeStruct((_E, _HID), jnp.float32)] * 2,
        mesh=plsc.VectorSubcoreMesh(core_axis_name="c", subcore_axis_name="s"),
        scratch_types=buf_set + buf_set,
    )
    return f(x, dst, src)


def _edge_body(e_ref, xd_ref, xs_ref, wqT, wkvT, qb, kvb, ewT, ewb, A_ref,
               c1T, c1b, c2T, c2b, S_ref, l1w, l1b, l2w, l2b,
               ed_ref, co_ref):
    e_blk = e_ref[...]
    eh = jnp.dot(e_blk, ewT[...], preferred_element_type=jnp.float32)
    eh = eh + ewb[...]
    Ew = eh[:, :_HID]
    Eb = eh[:, _HID:]
    kv = jnp.dot(xs_ref[...], wkvT[...], preferred_element_type=jnp.float32)
    kv = kv + kvb[...]
    qk = (jnp.dot(xd_ref[...], wqT[...], preferred_element_type=jnp.float32)
          + kv[:, :_HID] + qb[...])
    conn1 = qk * Ew
    a = jnp.abs(conn1)
    conn2 = (jnp.sign(conn1) * jnp.sqrt(jnp.where(a > 0, a, 1.0))
             * (a > 0).astype(jnp.float32))
    conn = jnp.maximum(conn2 + Eb, 0.0)
    score = jnp.dot(conn, A_ref[...], preferred_element_type=jnp.float32)
    w = jnp.exp(jnp.clip(score, -_CLAMP, _CLAMP))
    cc1 = jnp.dot(conn, c1T[...], preferred_element_type=jnp.float32)
    cc1 = cc1 + c1b[...]
    msg = kv[:, _HID:] + cc1
    wfull = jnp.dot(w, S_ref[...], preferred_element_type=jnp.float32)
    ed_ref[:, :_HID] = msg * wfull
    ed_ref[:, _HID:_HID + _HEADS] = w
    ed_ref[:, _HID + _HEADS:] = jnp.zeros(
        (e_blk.shape[0], _EDW - _HID - _HEADS), jnp.float32)
    co = _ln_tc(cc1, l1w[...], l1b[...])
    co = jnp.maximum(co, 0.0)
    co = jnp.dot(co, c2T[...], preferred_element_type=jnp.float32)
    co = co + c2b[...] + e_blk
    co_ref[...] = _ln_tc(co, l2w[...], l2b[...])


def _edge_call(e, xd, xs, wqT, wkvT, qb, kvb, ewT, ewb, A, c1T, c1b, c2T,
               c2b, S, l1w, l1b, l2w, l2b):
    full = lambda shape: pl.BlockSpec(shape, lambda i: (0, 0))
    return pl.pallas_call(
        _edge_body,
        grid=(_E // _EB,),
        in_specs=[
            pl.BlockSpec((_EB, _HID), lambda i: (i, 0)),
            pl.BlockSpec((_EB, _HID), lambda i: (i, 0)),
            pl.BlockSpec((_EB, _HID), lambda i: (i, 0)),
            full((_HID, _HID)),
            full((_HID, 2 * _HID)),
            full((1, _HID)),
            full((1, 2 * _HID)),
            full((_HID, 2 * _HID)),
            full((1, 2 * _HID)),
            full((_HID, _HEADS)),
            full((_HID, _HID)),
            full((1, _HID)),
            full((_HID, _HID)),
            full((1, _HID)),
            full((_HEADS, _HID)),
            full((1, _HID)),
            full((1, _HID)),
            full((1, _HID)),
            full((1, _HID)),
        ],
        out_specs=[
            pl.BlockSpec((_EB, _EDW), lambda i: (i, 0)),
            pl.BlockSpec((_EB, _HID), lambda i: (i, 0)),
        ],
        out_shape=[
            jax.ShapeDtypeStruct((_E, _EDW), jnp.float32),
            jax.ShapeDtypeStruct((_E, _HID), jnp.float32),
        ],
    )(e, xd, xs, wqT, wkvT, qb, kvb, ewT, ewb, A, c1T, c1b, c2T, c2b, S,
      l1w, l1b, l2w, l2b)


def _scatter_body(ed_hbm, dst_hbm, out_hbm, idx0, rows0, sem0,
                  idx1, rows1, sem1, zb, acc):
    bufs = ((idx0, rows0, sem0), (idx1, rows1, sem1))
    cid = lax.axis_index("c")
    sid = lax.axis_index("s")
    z16 = jnp.zeros((16,), jnp.float32)

    def zrow(r, c):
        for cc in range(_EDW // 16):
            zb[r, pl.ds(cc * 16, 16)] = z16
        return c

    lax.fori_loop(0, _ROWS_T, zrow, 0)
    pltpu.sync_copy(zb, acc.at[pl.ds(sid * _ROWS_T, _ROWS_T)])
    plsc.subcore_barrier()

    lo = cid * _HN
    trash = _HN + sid
    base = sid * _PERW_S

    def issue(j, buf):
        idxv, rows, sem = buf
        off = base + j * _CG
        pltpu.sync_copy(dst_hbm.at[pl.ds(off, _CG)], idxv)
        for s in range(_CG // 16):
            sl = pl.ds(s * 16, 16)
            v = idxv[sl]
            local = v - lo
            ok = (local >= 0) & (local < _HN)
            idxv[sl] = jnp.where(ok, local, trash)
        pltpu.async_copy(ed_hbm.at[pl.ds(off, _CG)], rows, sem)

    def finish(j, buf):
        idxv, rows, sem = buf
        off = base + j * _CG
        pltpu.make_async_copy(ed_hbm.at[pl.ds(off, _CG)], rows, sem).wait()
        pltpu.sync_copy(rows, acc.at[idxv], add=True)

    issue(0, bufs[0])

    def outer(g, carry):
        j0 = 2 * g
        issue(j0 + 1, bufs[1])
        finish(j0, bufs[0])
        issue(j0 + 2, bufs[0])
        finish(j0 + 1, bufs[1])
        return carry

    lax.fori_loop(0, _NCHS // 2 - 1, outer, 0)
    issue(_NCHS - 1, bufs[1])
    finish(_NCHS - 2, bufs[0])
    finish(_NCHS - 1, bufs[1])
    plsc.subcore_barrier()
    pltpu.sync_copy(acc.at[pl.ds(sid * _ROWS_T, _ROWS_T)],
                    out_hbm.at[cid, pl.ds(sid * _ROWS_T, _ROWS_T)])


def _scatter_call(ed, dst):
    buf_set = [
        pltpu.VMEM((_CG,), jnp.int32),
        pltpu.VMEM((_CG, _EDW), jnp.float32),
        pltpu.SemaphoreType.DMA,
    ]
    f = pl.kernel(
        _scatter_body,
        out_type=jax.ShapeDtypeStruct((_NC, _ACC_R, _EDW), jnp.float32),
        mesh=plsc.VectorSubcoreMesh(core_axis_name="c", subcore_axis_name="s"),
        scratch_types=buf_set + buf_set + [
            pltpu.VMEM((_ROWS_T, _EDW), jnp.float32),
            pltpu.VMEM_SHARED((_ACC_R, _EDW), jnp.float32),
        ],
        compiler_params=pltpu.CompilerParams(use_tc_tiling_on_sc=False),
    )
    return f(ed, dst)


def _node_body(x_ref, p_ref, S_ref, f1T, f1b, f2T, f2b, l1w, l1b, l2w, l2b,
               h_ref):
    p = p_ref[0]
    aggW = p[:, :_HID]
    sumw = p[:, _HID:_HID + _HEADS]
    inv = 1.0 / (sumw + 1e-16)
    agg = aggW * jnp.dot(inv, S_ref[...], preferred_element_type=jnp.float32)
    h0 = x_ref[...] + agg
    h = _ln_tc(h0, l1w[...], l1b[...])
    h = jnp.dot(h, f1T[...], preferred_element_type=jnp.float32) + f1b[...]
    h = jnp.maximum(h, 0.0)
    h = jnp.dot(h, f2T[...], preferred_element_type=jnp.float32) + f2b[...]
    h = h + h0
    h_ref[...] = _ln_tc(h, l2w[...], l2b[...])


def _node_call(x, parts, S, f1T, f1b, f2T, f2b, l1w, l1b, l2w, l2b):
    full = lambda shape: pl.BlockSpec(shape, lambda i: (0,) * len(shape))
    return pl.pallas_call(
        _node_body,
        grid=(_N // _NB,),
        in_specs=[
            pl.BlockSpec((_NB, _HID), lambda i: (i, 0)),
            pl.BlockSpec((1, _NB, _EDW),
                         lambda i: (i // (_HN // _NB), i % (_HN // _NB), 0)),
            full((_HEADS, _HID)),
            full((_HID, 2 * _HID)),
            full((1, 2 * _HID)),
            full((2 * _HID, _HID)),
            full((1, _HID)),
            full((1, _HID)),
            full((1, _HID)),
            full((1, _HID)),
            full((1, _HID)),
        ],
        out_specs=pl.BlockSpec((_NB, _HID), lambda i: (i, 0)),
        out_shape=jax.ShapeDtypeStruct((_N, _HID), jnp.float32),
    )(x, parts, S, f1T, f1b, f2T, f2b, l1w, l1b, l2w, l2b)


def kernel(x, e, edge_index, qkv_w, qkv_b, ew_w, ew_b, aw, c1_w, c1_b,
           c2_w, c2_b, f1_w, f1_b, f2_w, f2_b, ln1h_w, ln1h_b, ln2h_w,
           ln2h_b, ln1c_w, ln1c_b, ln2c_w, ln2c_b):
    f32 = jnp.float32
    dst = edge_index[0]
    src = edge_index[1]
    row = lambda b: b.reshape(1, -1).astype(f32)

    S = jnp.repeat(jnp.eye(_HEADS, dtype=f32), _DH, axis=1)
    awflat = aw[:, :, 0].T.reshape(_HID, 1)
    A = S.T * awflat

    wqT = qkv_w[:_HID].T
    wkvT = qkv_w[_HID:].T
    xd, xs = _gather_call(x, dst, src)
    ed, co = _edge_call(e, xd, xs, wqT, wkvT, row(qkv_b[:_HID]),
                        row(qkv_b[_HID:]), ew_w.T, row(ew_b), A,
                        c1_w.T, row(c1_b), c2_w.T, row(c2_b), S,
                        row(ln1c_w), row(ln1c_b), row(ln2c_w), row(ln2c_b))
    parts = _scatter_call(ed, dst)
    h = _node_call(x, parts, S, f1_w.T, row(f1_b), f2_w.T, row(f2_b),
                   row(ln1h_w), row(ln1h_b), row(ln2h_w), row(ln2h_b))
    return h, co


# speedup vs baseline: 34.2912x; 1.1562x over previous
import functools

import jax
import jax.numpy as jnp
from jax import lax
from jax.experimental import pallas as pl
from jax.experimental.pallas import tpu as pltpu
from jax.experimental.pallas import tpu_sc as plsc

_N = 10000
_E = 320000
_HID = 128
_HEADS = 8
_DH = 16
_CLAMP = 5.0
_EPS = 1e-5

_NC = 2
_NS = 16
_NW = _NC * _NS
_PERW = _E // _NW
_CG = 80
_NCHG = _PERW // _CG
_SW = 16
_HN = _N // _NC
_ACC_R = 5120
_ROWS_T = _ACC_R // _NS
_PERW_S = _E // _NS
_NCHS = _PERW_S // _CG

_NB = 1000
_EB = 2000


def _ln_tc(h, w, b):
    mu = jnp.mean(h, axis=-1, keepdims=True)
    var = jnp.mean((h - mu) ** 2, axis=-1, keepdims=True)
    return (h - mu) * jax.lax.rsqrt(var + _EPS) * w + b


def _gather_body(x_hbm, dst_hbm, src_hbm, xd_hbm, xs_hbm,
                 idxd0, idxs0, db0, sb0, sem0,
                 idxd1, idxs1, db1, sb1, sem1):
    bufs = ((idxd0, idxs0, db0, sb0, sem0),
            (idxd1, idxs1, db1, sb1, sem1))
    wid = lax.axis_index("s") * _NC + lax.axis_index("c")
    base = wid * _PERW

    def issue(j, buf):
        idxd, idxs, db, sb, sem = buf
        off = base + j * _CG
        pltpu.sync_copy(dst_hbm.at[pl.ds(off, _CG)], idxd)
        pltpu.sync_copy(src_hbm.at[pl.ds(off, _CG)], idxs)
        pltpu.async_copy(x_hbm.at[idxd], db, sem)
        pltpu.async_copy(x_hbm.at[idxs], sb, sem)

    def finish(j, buf):
        idxd, idxs, db, sb, sem = buf
        off = base + j * _CG
        pltpu.make_async_copy(x_hbm.at[idxd], db, sem).wait()
        pltpu.make_async_copy(x_hbm.at[idxs], sb, sem).wait()
        pltpu.sync_copy(db, xd_hbm.at[pl.ds(off, _CG)])
        pltpu.sync_copy(sb, xs_hbm.at[pl.ds(off, _CG)])

    issue(0, bufs[0])

    def outer(g, carry):
        j0 = 2 * g
        issue(j0 + 1, bufs[1])
        finish(j0, bufs[0])
        issue(j0 + 2, bufs[0])
        finish(j0 + 1, bufs[1])
        return carry

    lax.fori_loop(0, _NCHG // 2, outer, 0)
    finish(_NCHG - 1, bufs[0])


def _gather_call(x, dst, src):
    buf_set = [
        pltpu.VMEM((_CG,), jnp.int32),
        pltpu.VMEM((_CG,), jnp.int32),
        pltpu.VMEM((_CG, _HID), jnp.float32),
        pltpu.VMEM((_CG, _HID), jnp.float32),
        pltpu.SemaphoreType.DMA,
    ]
    f = pl.kernel(
        _gather_body,
        out_type=[jax.ShapeDtypeStruct((_E, _HID), jnp.float32)] * 2,
        mesh=plsc.VectorSubcoreMesh(core_axis_name="c", subcore_axis_name="s"),
        scratch_types=buf_set + buf_set,
    )
    return f(x, dst, src)


def _edge_body(e_ref, xd_ref, xs_ref, wqT, wkvT, qb, kvb, ewT, ewb, A_ref,
               c1T, c1b, S_ref, edw_ref, eds_ref, cc1_ref):
    e_blk = e_ref[...]
    eh = jnp.dot(e_blk, ewT[...], preferred_element_type=jnp.float32)
    eh = eh + ewb[...]
    Ew = eh[:, :_HID]
    Eb = eh[:, _HID:]
    kv = jnp.dot(xs_ref[...], wkvT[...], preferred_element_type=jnp.float32)
    kv = kv + kvb[...]
    qk = (jnp.dot(xd_ref[...], wqT[...], preferred_element_type=jnp.float32)
          + kv[:, :_HID] + qb[...])
    conn1 = qk * Ew
    a = jnp.abs(conn1)
    conn2 = (jnp.sign(conn1) * jnp.sqrt(jnp.where(a > 0, a, 1.0))
             * (a > 0).astype(jnp.float32))
    conn = jnp.maximum(conn2 + Eb, 0.0)
    score = jnp.dot(conn, A_ref[...], preferred_element_type=jnp.float32)
    w = jnp.exp(jnp.clip(score, -_CLAMP, _CLAMP))
    cc1 = jnp.dot(conn, c1T[...], preferred_element_type=jnp.float32)
    cc1 = cc1 + c1b[...]
    msg = kv[:, _HID:] + cc1
    wfull = jnp.dot(w, S_ref[...], preferred_element_type=jnp.float32)
    edw_ref[...] = msg * wfull
    eds_ref[:, :_HEADS] = w
    eds_ref[:, _HEADS:] = jnp.zeros((e_blk.shape[0], _SW - _HEADS),
                                    jnp.float32)
    cc1_ref[...] = cc1


def _edge_call(e, xd, xs, wqT, wkvT, qb, kvb, ewT, ewb, A, c1T, c1b, S):
    full = lambda shape: pl.BlockSpec(shape, lambda i: (0, 0))
    return pl.pallas_call(
        _edge_body,
        grid=(_E // _EB,),
        in_specs=[
            pl.BlockSpec((_EB, _HID), lambda i: (i, 0)),
            pl.BlockSpec((_EB, _HID), lambda i: (i, 0)),
            pl.BlockSpec((_EB, _HID), lambda i: (i, 0)),
            full((_HID, _HID)),
            full((_HID, 2 * _HID)),
            full((1, _HID)),
            full((1, 2 * _HID)),
            full((_HID, 2 * _HID)),
            full((1, 2 * _HID)),
            full((_HID, _HEADS)),
            full((_HID, _HID)),
            full((1, _HID)),
            full((_HEADS, _HID)),
        ],
        out_specs=[
            pl.BlockSpec((_EB, _HID), lambda i: (i, 0)),
            pl.BlockSpec((_EB, _SW), lambda i: (i, 0)),
            pl.BlockSpec((_EB, _HID), lambda i: (i, 0)),
        ],
        out_shape=[
            jax.ShapeDtypeStruct((_E, _HID), jnp.float32),
            jax.ShapeDtypeStruct((_E, _SW), jnp.float32),
            jax.ShapeDtypeStruct((_E, _HID), jnp.float32),
        ],
    )(e, xd, xs, wqT, wkvT, qb, kvb, ewT, ewb, A, c1T, c1b, S)


def _conn_body(e_ref, cc1_ref, c2T, c2b, l1w, l1b, l2w, l2b, co_ref):
    cc1 = cc1_ref[...]
    co = _ln_tc(cc1, l1w[...], l1b[...])
    co = jnp.maximum(co, 0.0)
    co = jnp.dot(co, c2T[...], preferred_element_type=jnp.float32)
    co = co + c2b[...] + e_ref[...]
    co_ref[...] = _ln_tc(co, l2w[...], l2b[...])


def _conn_call(e, cc1, c2T, c2b, l1w, l1b, l2w, l2b):
    full = lambda shape: pl.BlockSpec(shape, lambda i: (0, 0))
    return pl.pallas_call(
        _conn_body,
        grid=(_E // _EB,),
        in_specs=[
            pl.BlockSpec((_EB, _HID), lambda i: (i, 0)),
            pl.BlockSpec((_EB, _HID), lambda i: (i, 0)),
            full((_HID, _HID)),
            full((1, _HID)),
            full((1, _HID)),
            full((1, _HID)),
            full((1, _HID)),
            full((1, _HID)),
        ],
        out_specs=pl.BlockSpec((_EB, _HID), lambda i: (i, 0)),
        out_shape=jax.ShapeDtypeStruct((_E, _HID), jnp.float32),
    )(e, cc1, c2T, c2b, l1w, l1b, l2w, l2b)


def _scatter_body(edw_hbm, eds_hbm, dst_hbm, outw_hbm, outs_hbm,
                  idx0, rw0, rs0, sem0, idx1, rw1, rs1, sem1,
                  zbw, zbs, accw, accs):
    bufs = ((idx0, rw0, rs0, sem0), (idx1, rw1, rs1, sem1))
    cid = lax.axis_index("c")
    sid = lax.axis_index("s")
    z16 = jnp.zeros((16,), jnp.float32)

    def zrow(r, c):
        for cc in range(_HID // 16):
            zbw[r, pl.ds(cc * 16, 16)] = z16
        zbs[r, :] = z16
        return c

    lax.fori_loop(0, _ROWS_T, zrow, 0)
    pltpu.sync_copy(zbw, accw.at[pl.ds(sid * _ROWS_T, _ROWS_T)])
    pltpu.sync_copy(zbs, accs.at[pl.ds(sid * _ROWS_T, _ROWS_T)])
    plsc.subcore_barrier()

    lo = cid * _HN
    trash = _HN + sid
    base = sid * _PERW_S

    def issue(j, buf):
        idxv, rw, rs, sem = buf
        off = base + j * _CG
        pltpu.sync_copy(dst_hbm.at[pl.ds(off, _CG)], idxv)
        for s in range(_CG // 16):
            sl = pl.ds(s * 16, 16)
            v = idxv[sl]
            local = v - lo
            ok = (local >= 0) & (local < _HN)
            idxv[sl] = jnp.where(ok, local, trash)
        pltpu.async_copy(edw_hbm.at[pl.ds(off, _CG)], rw, sem)
        pltpu.async_copy(eds_hbm.at[pl.ds(off, _CG)], rs, sem)

    def finish(j, buf):
        idxv, rw, rs, sem = buf
        off = base + j * _CG
        pltpu.make_async_copy(edw_hbm.at[pl.ds(off, _CG)], rw, sem).wait()
        pltpu.make_async_copy(eds_hbm.at[pl.ds(off, _CG)], rs, sem).wait()
        pltpu.sync_copy(rw, accw.at[idxv], add=True)
        pltpu.sync_copy(rs, accs.at[idxv], add=True)

    issue(0, bufs[0])

    def outer(g, carry):
        j0 = 2 * g
        issue(j0 + 1, bufs[1])
        finish(j0, bufs[0])
        issue(j0 + 2, bufs[0])
        finish(j0 + 1, bufs[1])
        return carry

    lax.fori_loop(0, _NCHS // 2 - 1, outer, 0)
    issue(_NCHS - 1, bufs[1])
    finish(_NCHS - 2, bufs[0])
    finish(_NCHS - 1, bufs[1])
    plsc.subcore_barrier()
    pltpu.sync_copy(accw.at[pl.ds(sid * _ROWS_T, _ROWS_T)],
                    outw_hbm.at[cid, pl.ds(sid * _ROWS_T, _ROWS_T)])
    pltpu.sync_copy(accs.at[pl.ds(sid * _ROWS_T, _ROWS_T)],
                    outs_hbm.at[cid, pl.ds(sid * _ROWS_T, _ROWS_T)])


def _scatter_call(edw, eds, dst):
    buf_set = [
        pltpu.VMEM((_CG,), jnp.int32),
        pltpu.VMEM((_CG, _HID), jnp.float32),
        pltpu.VMEM((_CG, _SW), jnp.float32),
        pltpu.SemaphoreType.DMA,
    ]
    f = pl.kernel(
        _scatter_body,
        out_type=[
            jax.ShapeDtypeStruct((_NC, _ACC_R, _HID), jnp.float32),
            jax.ShapeDtypeStruct((_NC, _ACC_R, _SW), jnp.float32),
        ],
        mesh=plsc.VectorSubcoreMesh(core_axis_name="c", subcore_axis_name="s"),
        scratch_types=buf_set + buf_set + [
            pltpu.VMEM((_ROWS_T, _HID), jnp.float32),
            pltpu.VMEM((_ROWS_T, _SW), jnp.float32),
            pltpu.VMEM_SHARED((_ACC_R, _HID), jnp.float32),
            pltpu.VMEM_SHARED((_ACC_R, _SW), jnp.float32),
        ],
        compiler_params=pltpu.CompilerParams(use_tc_tiling_on_sc=False),
    )
    return f(edw, eds, dst)


def _node_body(x_ref, pw_ref, ps_ref, S_ref, f1T, f1b, f2T, f2b, l1w, l1b,
               l2w, l2b, h_ref):
    aggW = pw_ref[0]
    sumw = ps_ref[0][:, :_HEADS]
    inv = 1.0 / (sumw + 1e-16)
    agg = aggW * jnp.dot(inv, S_ref[...], preferred_element_type=jnp.float32)
    h0 = x_ref[...] + agg
    h = _ln_tc(h0, l1w[...], l1b[...])
    h = jnp.dot(h, f1T[...], preferred_element_type=jnp.float32) + f1b[...]
    h = jnp.maximum(h, 0.0)
    h = jnp.dot(h, f2T[...], preferred_element_type=jnp.float32) + f2b[...]
    h = h + h0
    h_ref[...] = _ln_tc(h, l2w[...], l2b[...])


def _node_call(x, pw, ps, S, f1T, f1b, f2T, f2b, l1w, l1b, l2w, l2b):
    full = lambda shape: pl.BlockSpec(shape, lambda i: (0,) * len(shape))
    return pl.pallas_call(
        _node_body,
        grid=(_N // _NB,),
        in_specs=[
            pl.BlockSpec((_NB, _HID), lambda i: (i, 0)),
            pl.BlockSpec((1, _NB, _HID),
                         lambda i: (i // (_HN // _NB), i % (_HN // _NB), 0)),
            pl.BlockSpec((1, _NB, _SW),
                         lambda i: (i // (_HN // _NB), i % (_HN // _NB), 0)),
            full((_HEADS, _HID)),
            full((_HID, 2 * _HID)),
            full((1, 2 * _HID)),
            full((2 * _HID, _HID)),
            full((1, _HID)),
            full((1, _HID)),
            full((1, _HID)),
            full((1, _HID)),
            full((1, _HID)),
        ],
        out_specs=pl.BlockSpec((_NB, _HID), lambda i: (i, 0)),
        out_shape=jax.ShapeDtypeStruct((_N, _HID), jnp.float32),
    )(x, pw, ps, S, f1T, f1b, f2T, f2b, l1w, l1b, l2w, l2b)


def kernel(x, e, edge_index, qkv_w, qkv_b, ew_w, ew_b, aw, c1_w, c1_b,
           c2_w, c2_b, f1_w, f1_b, f2_w, f2_b, ln1h_w, ln1h_b, ln2h_w,
           ln2h_b, ln1c_w, ln1c_b, ln2c_w, ln2c_b):
    f32 = jnp.float32
    dst = edge_index[0]
    src = edge_index[1]
    row = lambda b: b.reshape(1, -1).astype(f32)

    S = jnp.repeat(jnp.eye(_HEADS, dtype=f32), _DH, axis=1)
    awflat = aw[:, :, 0].T.reshape(_HID, 1)
    A = S.T * awflat

    wqT = qkv_w[:_HID].T
    wkvT = qkv_w[_HID:].T
    xd, xs = _gather_call(x, dst, src)
    edw, eds, cc1 = _edge_call(e, xd, xs, wqT, wkvT, row(qkv_b[:_HID]),
                               row(qkv_b[_HID:]), ew_w.T, row(ew_b), A,
                               c1_w.T, row(c1_b), S)
    pw, ps = _scatter_call(edw, eds, dst)
    co = _conn_call(e, cc1, c2_w.T, row(c2_b), row(ln1c_w), row(ln1c_b),
                    row(ln2c_w), row(ln2c_b))
    h = _node_call(x, pw, ps, S, f1_w.T, row(f1_b), f2_w.T, row(f2_b),
                   row(ln1h_w), row(ln1h_b), row(ln2h_w), row(ln2h_b))
    return h, co


# speedup vs baseline: 37.5093x; 1.0938x over previous
import functools

import jax
import jax.numpy as jnp
from jax import lax
from jax.experimental import pallas as pl
from jax.experimental.pallas import tpu as pltpu
from jax.experimental.pallas import tpu_sc as plsc

_N = 10000
_E = 320000
_HID = 128
_HEADS = 8
_DH = 16
_CLAMP = 5.0
_EPS = 1e-5

_NC = 2
_NS = 16
_NW = _NC * _NS
_PERW = _E // _NW
_CG = 80
_NCHG = _PERW // _CG
_SW = 16
_HN = _N // _NC
_ACC_R = 5120
_ROWS_T = _ACC_R // _NS
_PERW_S = _E // _NS
_NCHS = _PERW_S // _CG

_NB = 1000
_EB = 2000


def _ln_tc(h, w, b):
    mu = jnp.mean(h, axis=-1, keepdims=True)
    var = jnp.mean((h - mu) ** 2, axis=-1, keepdims=True)
    return (h - mu) * jax.lax.rsqrt(var + _EPS) * w + b


def _gather_body(x_hbm, dst3_hbm, src3_hbm, xd_hbm, xs_hbm,
                 idxd, idxs, db0, sb0, sem0, db1, sb1, sem1):
    bufs = ((db0, sb0, sem0), (db1, sb1, sem1))
    wid = lax.axis_index("s") * _NC + lax.axis_index("c")
    base = wid * _PERW
    pltpu.sync_copy(dst3_hbm.at[wid], idxd)
    pltpu.sync_copy(src3_hbm.at[wid], idxs)

    def issue(j, buf):
        db, sb, sem = buf
        pltpu.async_copy(x_hbm.at[idxd.at[j]], db, sem)
        pltpu.async_copy(x_hbm.at[idxs.at[j]], sb, sem)

    def finish(j, buf):
        db, sb, sem = buf
        off = base + j * _CG
        pltpu.make_async_copy(x_hbm.at[idxd.at[j]], db, sem).wait()
        pltpu.make_async_copy(x_hbm.at[idxs.at[j]], sb, sem).wait()
        pltpu.sync_copy(db, xd_hbm.at[pl.ds(off, _CG)])
        pltpu.sync_copy(sb, xs_hbm.at[pl.ds(off, _CG)])

    issue(0, bufs[0])

    def outer(g, carry):
        j0 = 2 * g
        issue(j0 + 1, bufs[1])
        finish(j0, bufs[0])
        issue(j0 + 2, bufs[0])
        finish(j0 + 1, bufs[1])
        return carry

    lax.fori_loop(0, _NCHG // 2, outer, 0)
    finish(_NCHG - 1, bufs[0])


def _gather_call(x, dst3, src3):
    buf_set = [
        pltpu.VMEM((_CG, _HID), jnp.float32),
        pltpu.VMEM((_CG, _HID), jnp.float32),
        pltpu.SemaphoreType.DMA,
    ]
    f = pl.kernel(
        _gather_body,
        out_type=[jax.ShapeDtypeStruct((_E, _HID), jnp.float32)] * 2,
        mesh=plsc.VectorSubcoreMesh(core_axis_name="c", subcore_axis_name="s"),
        scratch_types=[
            pltpu.VMEM((_NCHG, _CG), jnp.int32),
            pltpu.VMEM((_NCHG, _CG), jnp.int32),
        ] + buf_set + buf_set,
    )
    return f(x, dst3, src3)


def _edge_body(e_ref, xd_ref, xs_ref, wqT, wkvT, qb, kvb, ewT, ewb, A_ref,
               c1T, c1b, S_ref, edw_ref, eds_ref, cc1_ref):
    e_blk = e_ref[...]
    eh = jnp.dot(e_blk, ewT[...], preferred_element_type=jnp.float32)
    eh = eh + ewb[...]
    Ew = eh[:, :_HID]
    Eb = eh[:, _HID:]
    kv = jnp.dot(xs_ref[...], wkvT[...], preferred_element_type=jnp.float32)
    kv = kv + kvb[...]
    qk = (jnp.dot(xd_ref[...], wqT[...], preferred_element_type=jnp.float32)
          + kv[:, :_HID] + qb[...])
    conn1 = qk * Ew
    a = jnp.abs(conn1)
    conn2 = (jnp.sign(conn1) * jnp.sqrt(jnp.where(a > 0, a, 1.0))
             * (a > 0).astype(jnp.float32))
    conn = jnp.maximum(conn2 + Eb, 0.0)
    score = jnp.dot(conn, A_ref[...], preferred_element_type=jnp.float32)
    w = jnp.exp(jnp.clip(score, -_CLAMP, _CLAMP))
    cc1 = jnp.dot(conn, c1T[...], preferred_element_type=jnp.float32)
    cc1 = cc1 + c1b[...]
    msg = kv[:, _HID:] + cc1
    wfull = jnp.dot(w, S_ref[...], preferred_element_type=jnp.float32)
    edw_ref[...] = msg * wfull
    eds_ref[:, :_HEADS] = w
    eds_ref[:, _HEADS:] = jnp.zeros((e_blk.shape[0], _SW - _HEADS),
                                    jnp.float32)
    cc1_ref[...] = cc1


def _edge_call(e, xd, xs, wqT, wkvT, qb, kvb, ewT, ewb, A, c1T, c1b, S):
    full = lambda shape: pl.BlockSpec(shape, lambda i: (0, 0))
    return pl.pallas_call(
        _edge_body,
        grid=(_E // _EB,),
        in_specs=[
            pl.BlockSpec((_EB, _HID), lambda i: (i, 0)),
            pl.BlockSpec((_EB, _HID), lambda i: (i, 0)),
            pl.BlockSpec((_EB, _HID), lambda i: (i, 0)),
            full((_HID, _HID)),
            full((_HID, 2 * _HID)),
            full((1, _HID)),
            full((1, 2 * _HID)),
            full((_HID, 2 * _HID)),
            full((1, 2 * _HID)),
            full((_HID, _HEADS)),
            full((_HID, _HID)),
            full((1, _HID)),
            full((_HEADS, _HID)),
        ],
        out_specs=[
            pl.BlockSpec((_EB, _HID), lambda i: (i, 0)),
            pl.BlockSpec((_EB, _SW), lambda i: (i, 0)),
            pl.BlockSpec((_EB, _HID), lambda i: (i, 0)),
        ],
        out_shape=[
            jax.ShapeDtypeStruct((_E, _HID), jnp.float32),
            jax.ShapeDtypeStruct((_E, _SW), jnp.float32),
            jax.ShapeDtypeStruct((_E, _HID), jnp.float32),
        ],
    )(e, xd, xs, wqT, wkvT, qb, kvb, ewT, ewb, A, c1T, c1b, S)


def _conn_body(e_ref, cc1_ref, c2T, c2b, l1w, l1b, l2w, l2b, co_ref):
    cc1 = cc1_ref[...]
    co = _ln_tc(cc1, l1w[...], l1b[...])
    co = jnp.maximum(co, 0.0)
    co = jnp.dot(co, c2T[...], preferred_element_type=jnp.float32)
    co = co + c2b[...] + e_ref[...]
    co_ref[...] = _ln_tc(co, l2w[...], l2b[...])


def _conn_call(e, cc1, c2T, c2b, l1w, l1b, l2w, l2b):
    full = lambda shape: pl.BlockSpec(shape, lambda i: (0, 0))
    return pl.pallas_call(
        _conn_body,
        grid=(_E // _EB,),
        in_specs=[
            pl.BlockSpec((_EB, _HID), lambda i: (i, 0)),
            pl.BlockSpec((_EB, _HID), lambda i: (i, 0)),
            full((_HID, _HID)),
            full((1, _HID)),
            full((1, _HID)),
            full((1, _HID)),
            full((1, _HID)),
            full((1, _HID)),
        ],
        out_specs=pl.BlockSpec((_EB, _HID), lambda i: (i, 0)),
        out_shape=jax.ShapeDtypeStruct((_E, _HID), jnp.float32),
    )(e, cc1, c2T, c2b, l1w, l1b, l2w, l2b)


def _scatter_body(edw_hbm, eds_hbm, dst3_hbm, outw_hbm, outs_hbm,
                  idx0, rw0, rs0, sem0, idx1, rw1, rs1, sem1,
                  zbw, zbs, accw, accs):
    bufs = ((idx0, rw0, rs0, sem0), (idx1, rw1, rs1, sem1))
    cid = lax.axis_index("c")
    sid = lax.axis_index("s")
    z16 = jnp.zeros((16,), jnp.float32)

    lo = cid * _HN
    trash = _HN + sid

    def zrow(r, c):
        for cc in range(_HID // 16):
            zbw[r, pl.ds(cc * 16, 16)] = z16
        zbs[r, :] = z16
        return c

    lax.fori_loop(0, _ROWS_T, zrow, 0)
    pltpu.sync_copy(zbw, accw.at[pl.ds(sid * _ROWS_T, _ROWS_T)])
    pltpu.sync_copy(zbs, accs.at[pl.ds(sid * _ROWS_T, _ROWS_T)])
    plsc.subcore_barrier()

    base = sid * _PERW_S

    def issue(j, buf):
        idxv, rw, rs, sem = buf
        off = base + j * _CG
        pltpu.async_copy(dst3_hbm.at[sid, j], idxv, sem)
        pltpu.async_copy(edw_hbm.at[pl.ds(off, _CG)], rw, sem)
        pltpu.async_copy(eds_hbm.at[pl.ds(off, _CG)], rs, sem)

    def finish(j, buf):
        idxv, rw, rs, sem = buf
        off = base + j * _CG
        pltpu.make_async_copy(dst3_hbm.at[sid, j], idxv, sem).wait()
        pltpu.make_async_copy(edw_hbm.at[pl.ds(off, _CG)], rw, sem).wait()
        pltpu.make_async_copy(eds_hbm.at[pl.ds(off, _CG)], rs, sem).wait()
        for s in range(_CG // 16):
            sl = pl.ds(s * 16, 16)
            v = idxv[sl]
            local = v - lo
            ok = (local >= 0) & (local < _HN)
            idxv[sl] = jnp.where(ok, local, trash)
        pltpu.sync_copy(rw, accw.at[idxv], add=True)
        pltpu.sync_copy(rs, accs.at[idxv], add=True)

    issue(0, bufs[0])

    def outer(g, carry):
        j0 = 2 * g
        issue(j0 + 1, bufs[1])
        finish(j0, bufs[0])
        issue(j0 + 2, bufs[0])
        finish(j0 + 1, bufs[1])
        return carry

    lax.fori_loop(0, _NCHS // 2 - 1, outer, 0)
    issue(_NCHS - 1, bufs[1])
    finish(_NCHS - 2, bufs[0])
    finish(_NCHS - 1, bufs[1])
    plsc.subcore_barrier()
    pltpu.sync_copy(accw.at[pl.ds(sid * _ROWS_T, _ROWS_T)],
                    outw_hbm.at[cid, pl.ds(sid * _ROWS_T, _ROWS_T)])
    pltpu.sync_copy(accs.at[pl.ds(sid * _ROWS_T, _ROWS_T)],
                    outs_hbm.at[cid, pl.ds(sid * _ROWS_T, _ROWS_T)])


def _scatter_call(edw, eds, dst3s):
    buf_set = [
        pltpu.VMEM((_CG,), jnp.int32),
        pltpu.VMEM((_CG, _HID), jnp.float32),
        pltpu.VMEM((_CG, _SW), jnp.float32),
        pltpu.SemaphoreType.DMA,
    ]
    f = pl.kernel(
        _scatter_body,
        out_type=[
            jax.ShapeDtypeStruct((_NC, _ACC_R, _HID), jnp.float32),
            jax.ShapeDtypeStruct((_NC, _ACC_R, _SW), jnp.float32),
        ],
        mesh=plsc.VectorSubcoreMesh(core_axis_name="c", subcore_axis_name="s"),
        scratch_types=buf_set + buf_set + [
            pltpu.VMEM((_ROWS_T, _HID), jnp.float32),
            pltpu.VMEM((_ROWS_T, _SW), jnp.float32),
            pltpu.VMEM_SHARED((_ACC_R, _HID), jnp.float32),
            pltpu.VMEM_SHARED((_ACC_R, _SW), jnp.float32),
        ],
        compiler_params=pltpu.CompilerParams(use_tc_tiling_on_sc=False),
    )
    return f(edw, eds, dst3s)


def _node_body(x_ref, pw_ref, ps_ref, S_ref, f1T, f1b, f2T, f2b, l1w, l1b,
               l2w, l2b, h_ref):
    aggW = pw_ref[0]
    sumw = ps_ref[0][:, :_HEADS]
    inv = 1.0 / (sumw + 1e-16)
    agg = aggW * jnp.dot(inv, S_ref[...], preferred_element_type=jnp.float32)
    h0 = x_ref[...] + agg
    h = _ln_tc(h0, l1w[...], l1b[...])
    h = jnp.dot(h, f1T[...], preferred_element_type=jnp.float32) + f1b[...]
    h = jnp.maximum(h, 0.0)
    h = jnp.dot(h, f2T[...], preferred_element_type=jnp.float32) + f2b[...]
    h = h + h0
    h_ref[...] = _ln_tc(h, l2w[...], l2b[...])


def _node_call(x, pw, ps, S, f1T, f1b, f2T, f2b, l1w, l1b, l2w, l2b):
    full = lambda shape: pl.BlockSpec(shape, lambda i: (0,) * len(shape))
    return pl.pallas_call(
        _node_body,
        grid=(_N // _NB,),
        in_specs=[
            pl.BlockSpec((_NB, _HID), lambda i: (i, 0)),
            pl.BlockSpec((1, _NB, _HID),
                         lambda i: (i // (_HN // _NB), i % (_HN // _NB), 0)),
            pl.BlockSpec((1, _NB, _SW),
                         lambda i: (i // (_HN // _NB), i % (_HN // _NB), 0)),
            full((_HEADS, _HID)),
            full((_HID, 2 * _HID)),
            full((1, 2 * _HID)),
            full((2 * _HID, _HID)),
            full((1, _HID)),
            full((1, _HID)),
            full((1, _HID)),
            full((1, _HID)),
            full((1, _HID)),
        ],
        out_specs=pl.BlockSpec((_NB, _HID), lambda i: (i, 0)),
        out_shape=jax.ShapeDtypeStruct((_N, _HID), jnp.float32),
    )(x, pw, ps, S, f1T, f1b, f2T, f2b, l1w, l1b, l2w, l2b)


def kernel(x, e, edge_index, qkv_w, qkv_b, ew_w, ew_b, aw, c1_w, c1_b,
           c2_w, c2_b, f1_w, f1_b, f2_w, f2_b, ln1h_w, ln1h_b, ln2h_w,
           ln2h_b, ln1c_w, ln1c_b, ln2c_w, ln2c_b):
    f32 = jnp.float32
    dst = edge_index[0]
    src = edge_index[1]
    row = lambda b: b.reshape(1, -1).astype(f32)

    S = jnp.repeat(jnp.eye(_HEADS, dtype=f32), _DH, axis=1)
    awflat = aw[:, :, 0].T.reshape(_HID, 1)
    A = S.T * awflat

    wqT = qkv_w[:_HID].T
    wkvT = qkv_w[_HID:].T
    dst3 = dst.reshape(_NW, _NCHG, _CG)
    src3 = src.reshape(_NW, _NCHG, _CG)
    dst3s = dst.reshape(_NS, _NCHS, _CG)
    xd, xs = _gather_call(x, dst3, src3)
    edw, eds, cc1 = _edge_call(e, xd, xs, wqT, wkvT, row(qkv_b[:_HID]),
                               row(qkv_b[_HID:]), ew_w.T, row(ew_b), A,
                               c1_w.T, row(c1_b), S)
    pw, ps = _scatter_call(edw, eds, dst3s)
    co = _conn_call(e, cc1, c2_w.T, row(c2_b), row(ln1c_w), row(ln1c_b),
                    row(ln2c_w), row(ln2c_b))
    h = _node_call(x, pw, ps, S, f1_w.T, row(f1_b), f2_w.T, row(f2_b),
                   row(ln1h_w), row(ln1h_b), row(ln2h_w), row(ln2h_b))
    return h, co
